# baseline (speedup 1.0000x reference)
import functools

import jax
import jax.numpy as jnp
from jax import lax
from jax.experimental import pallas as pl
from jax.experimental.pallas import tpu as pltpu
from jax.experimental.pallas import tpu_sc as plsc

T = 2048
D = 1024
F = 2048
E = 8
K = 2
RB = 256
NRB = T // RB
BLK = 128
FB = 512
NF = F // FB
NS = T * K // BLK + E
P_PAD = NS * BLK


def _router_body(x_ref, gw_ref, logits_ref, rw_ref, sel_ref, rank_ref,
                 counts_ref, carry):
    i = pl.program_id(0)

    @pl.when(i == 0)
    def _():
        carry[...] = jnp.zeros_like(carry)

    x = x_ref[...]
    gw = gw_ref[...]
    logits = lax.dot_general(x, gw, (((1,), (1,)), ((), ())))
    logits_ref[...] = logits
    m = jnp.max(logits, axis=1, keepdims=True)
    ex = jnp.exp(logits - m)
    sm = ex / jnp.sum(ex, axis=1, keepdims=True)
    eio = lax.broadcasted_iota(jnp.int32, (RB, E), 1)
    m0 = jnp.max(sm, axis=1, keepdims=True)
    a0 = jnp.min(jnp.where(sm == m0, eio, E), axis=1, keepdims=True)
    sm1 = jnp.where(eio == a0, -jnp.inf, sm)
    m1 = jnp.max(sm1, axis=1, keepdims=True)
    a1 = jnp.min(jnp.where(sm1 == m1, eio, E), axis=1, keepdims=True)
    oh = (eio == a0).astype(jnp.float32) + (eio == a1).astype(jnp.float32)
    r0 = lax.broadcasted_iota(jnp.int32, (RB, RB), 0)
    r1 = lax.broadcasted_iota(jnp.int32, (RB, RB), 1)
    ltri = (r1 < r0).astype(jnp.float32)
    rank_f = lax.dot_general(ltri, oh, (((1,), (0,)), ((), ()))) + carry[...]
    rank0 = jnp.sum(jnp.where(eio == a0, rank_f, 0.0), axis=1, keepdims=True)
    rank1 = jnp.sum(jnp.where(eio == a1, rank_f, 0.0), axis=1, keepdims=True)
    rw_ref[...] = jnp.concatenate([m0, m1], axis=1)
    sel_ref[...] = jnp.concatenate([a0, a1], axis=1)
    rank_ref[...] = jnp.concatenate([rank0, rank1], axis=1).astype(jnp.int32)
    new_carry = carry[...] + jnp.sum(oh, axis=0, keepdims=True)
    carry[...] = new_carry

    @pl.when(i == NRB - 1)
    def _():
        counts_ref[...] = new_carry.astype(jnp.int32)


@jax.jit
def _router(x, gate_w):
    return pl.pallas_call(
        _router_body,
        grid=(NRB,),
        in_specs=[
            pl.BlockSpec((RB, D), lambda i: (i, 0)),
            pl.BlockSpec((E, D), lambda i: (0, 0)),
        ],
        out_specs=[
            pl.BlockSpec((RB, E), lambda i: (i, 0)),
            pl.BlockSpec((RB, K), lambda i: (i, 0)),
            pl.BlockSpec((RB, K), lambda i: (i, 0)),
            pl.BlockSpec((RB, K), lambda i: (i, 0)),
            pl.BlockSpec((1, E), lambda i: (0, 0)),
        ],
        out_shape=[
            jax.ShapeDtypeStruct((T, E), jnp.float32),
            jax.ShapeDtypeStruct((T, K), jnp.float32),
            jax.ShapeDtypeStruct((T, K), jnp.int32),
            jax.ShapeDtypeStruct((T, K), jnp.int32),
            jax.ShapeDtypeStruct((1, E), jnp.int32),
        ],
        scratch_shapes=[pltpu.VMEM((1, E), jnp.float32)],
    )(x, gate_w)


def _grouped_body(se_ref, valid_ref, rs_ref, ne_ref, sl_ref, hn_ref,
                  x_ref, wl_hbm, wv_hbm, w1_hbm, out_ref,
                  wlb, wvb, w1b, sem_l, sem_v, sem_1):
    s = pl.program_id(0)
    slot = sl_ref[s]

    def _issue(e_idx, slot_idx):
        pltpu.make_async_copy(wl_hbm.at[e_idx], wlb.at[slot_idx],
                              sem_l.at[slot_idx]).start()
        pltpu.make_async_copy(wv_hbm.at[e_idx], wvb.at[slot_idx],
                              sem_v.at[slot_idx]).start()
        pltpu.make_async_copy(w1_hbm.at[e_idx], w1b.at[slot_idx],
                              sem_1.at[slot_idx]).start()

    @pl.when(s == 0)
    def _():
        _issue(se_ref[0], slot)

    @pl.when(rs_ref[s] != 0)
    def _():
        @pl.when(hn_ref[s] != 0)
        def _():
            _issue(ne_ref[s], 1 - slot)

        e = se_ref[s]
        pltpu.make_async_copy(wl_hbm.at[e], wlb.at[slot],
                              sem_l.at[slot]).wait()
        pltpu.make_async_copy(wv_hbm.at[e], wvb.at[slot],
                              sem_v.at[slot]).wait()
        pltpu.make_async_copy(w1_hbm.at[e], w1b.at[slot],
                              sem_1.at[slot]).wait()

    @pl.when(valid_ref[s] != 0)
    def _():
        x = x_ref[...]
        wl = wlb[slot]
        wv = wvb[slot]
        w1 = w1b[slot]
        hl = lax.dot_general(x, wl, (((1,), (1,)), ((), ())))
        hv = lax.dot_general(x, wv, (((1,), (1,)), ((), ())))
        g = (0.5 * hl * (1.0 + lax.erf(hl * 0.7071067811865476))) * hv
        out_ref[...] = lax.dot_general(g, w1, (((1,), (1,)), ((), ())))


@jax.jit
def _grouped(step_expert, step_valid, run_start, next_expert, slot_par,
             has_next, x_s, w_lin, w_v, w_1):
    grid_spec = pltpu.PrefetchScalarGridSpec(
        num_scalar_prefetch=6,
        grid=(NS,),
        in_specs=[
            pl.BlockSpec((BLK, D), lambda s, *_: (s, 0)),
            pl.BlockSpec(memory_space=pl.ANY),
            pl.BlockSpec(memory_space=pl.ANY),
            pl.BlockSpec(memory_space=pl.ANY),
        ],
        out_specs=pl.BlockSpec((BLK, D), lambda s, *_: (s, 0)),
        scratch_shapes=[
            pltpu.VMEM((2, F, D), jnp.float32),
            pltpu.VMEM((2, F, D), jnp.float32),
            pltpu.VMEM((2, D, F), jnp.float32),
            pltpu.SemaphoreType.DMA((2,)),
            pltpu.SemaphoreType.DMA((2,)),
            pltpu.SemaphoreType.DMA((2,)),
        ],
    )
    return pl.pallas_call(
        _grouped_body,
        grid_spec=grid_spec,
        out_shape=jax.ShapeDtypeStruct((P_PAD, D), jnp.float32),
    )(step_expert, step_valid, run_start, next_expert, slot_par, has_next,
      x_s, w_lin, w_v, w_1)


NC = 2
NSUB = 16
NW = NC * NSUB
APW = T * K // NW
CH = 32
NCH = APW // CH
TPW = T // NW

@functools.cache
def _sc_mesh():
    return plsc.VectorSubcoreMesh(core_axis_name="c", subcore_axis_name="s")


def _dispatch_body(x_hbm, sel_hbm, rank_hbm, offs_hbm, xs_hbm, pos_hbm,
                   tok_scr, pos_scr, selv, rankv, offs_v, bufs, semg, sems):
    wid = lax.axis_index("s") * NC + lax.axis_index("c")
    base = wid * APW
    pltpu.sync_copy(offs_hbm, offs_v)
    pltpu.sync_copy(sel_hbm.at[pl.ds(base, APW)], selv)
    pltpu.sync_copy(rank_hbm.at[pl.ds(base, APW)], rankv)
    for j in range(NCH):
        for q in range(CH // 16):
            o = j * CH + q * 16
            sl = selv[pl.ds(o, 16)]
            rk = rankv[pl.ds(o, 16)]
            off = plsc.load_gather(offs_v, [sl])
            pos_scr[j, pl.ds(q * 16, 16)] = off + rk
            tvec = (base + o + lax.iota(jnp.int32, 16)) // K
            tok_scr[j, pl.ds(q * 16, 16)] = tvec
        pltpu.sync_copy(pos_scr.at[j], pos_hbm.at[pl.ds(base + j * CH, CH)])
    gh = [None] * NCH
    sh = [None] * NCH
    gh[0] = pltpu.async_copy(x_hbm.at[tok_scr.at[0]], bufs.at[0], semg.at[0])
    for j in range(NCH):
        b = j % 2
        gh[j].wait()
        sh[j] = pltpu.async_copy(bufs.at[b], xs_hbm.at[pos_scr.at[j]],
                                 sems.at[b])
        if j + 1 < NCH:
            if j >= 1:
                sh[j - 1].wait()
            gh[j + 1] = pltpu.async_copy(x_hbm.at[tok_scr.at[j + 1]],
                                         bufs.at[1 - b], semg.at[1 - b])
    sh[NCH - 2].wait()
    sh[NCH - 1].wait()


@jax.jit
def _dispatch(x, sel_f, rank_f, row_offs16):
    return pl.kernel(
        _dispatch_body,
        mesh=_sc_mesh(),
        out_type=[
            jax.ShapeDtypeStruct((P_PAD, D), jnp.float32),
            jax.ShapeDtypeStruct((T * K,), jnp.int32),
        ],
        scratch_types=[
            pltpu.VMEM((NCH, CH), jnp.int32),
            pltpu.VMEM((NCH, CH), jnp.int32),
            pltpu.VMEM((APW,), jnp.int32),
            pltpu.VMEM((APW,), jnp.int32),
            pltpu.VMEM((16,), jnp.int32),
            pltpu.VMEM((2, CH, D), jnp.float32),
            pltpu.SemaphoreType.DMA((2,)),
            pltpu.SemaphoreType.DMA((2,)),
        ],
        compiler_params=pltpu.CompilerParams(needs_layout_passes=False),
    )(x, sel_f, rank_f, row_offs16)


NCT = TPW // 16


def _combine_body(h_hbm, pos_hbm, rw_hbm, out_hbm,
                  posv, rwv, p0s, p1s, w0s, w1s, w0c, w1c,
                  bufa, bufb, bufo, sema, semb):
    wid = lax.axis_index("s") * NC + lax.axis_index("c")
    tbase = wid * TPW
    ab = tbase * K
    pltpu.sync_copy(pos_hbm.at[pl.ds(ab, TPW * K)], posv)
    pltpu.sync_copy(rw_hbm.at[pl.ds(ab, TPW * K)], rwv)
    ev = lax.iota(jnp.int32, 16) * 2
    od = ev + 1
    for j in range(NCT):
        o = j * 2 * 16
        p0s[j, :] = plsc.load_gather(posv, [o + ev])
        p1s[j, :] = plsc.load_gather(posv, [o + od])
        w0s[j, :] = plsc.load_gather(rwv, [o + ev])
        w1s[j, :] = plsc.load_gather(rwv, [o + od])
    ah = [None] * NCT
    bh = [None] * NCT
    ah[0] = pltpu.async_copy(h_hbm.at[p0s[0, :]], bufa.at[0], sema.at[0])
    bh[0] = pltpu.async_copy(h_hbm.at[p1s[0, :]], bufb.at[0], semb.at[0])
    for j in range(NCT):
        b = j % 2
        ah[j].wait()
        bh[j].wait()
        if j + 1 < NCT:
            ah[j + 1] = pltpu.async_copy(h_hbm.at[p0s[j + 1, :]],
                                         bufa.at[1 - b], sema.at[1 - b])
            bh[j + 1] = pltpu.async_copy(h_hbm.at[p1s[j + 1, :]],
                                         bufb.at[1 - b], semb.at[1 - b])
        w0c[...] = w0s[j, :]
        w1c[...] = w1s[j, :]

        def body(i, carry):
            iv = jnp.broadcast_to(i, (16,)).astype(jnp.int32)
            ws0 = plsc.load_gather(w0c, [iv])
            ws1 = plsc.load_gather(w1c, [iv])
            for q in range(D // 16):
                va = bufa[b, i, pl.ds(q * 16, 16)]
                vb = bufb[b, i, pl.ds(q * 16, 16)]
                bufo[i, pl.ds(q * 16, 16)] = va * ws0 + vb * ws1
            return carry

        lax.fori_loop(0, 16, body, 0)
        pltpu.sync_copy(bufo, out_hbm.at[pl.ds(tbase + j * 16, 16)])


@jax.jit
def _combine(h_s, pos, rw_f):
    return pl.kernel(
        _combine_body,
        mesh=_sc_mesh(),
        out_type=jax.ShapeDtypeStruct((T, D), jnp.float32),
        scratch_types=[
            pltpu.VMEM((TPW * K,), jnp.int32),
            pltpu.VMEM((TPW * K,), jnp.float32),
            pltpu.VMEM((NCT, 16), jnp.int32),
            pltpu.VMEM((NCT, 16), jnp.int32),
            pltpu.VMEM((NCT, 16), jnp.float32),
            pltpu.VMEM((NCT, 16), jnp.float32),
            pltpu.VMEM((16,), jnp.float32),
            pltpu.VMEM((16,), jnp.float32),
            pltpu.VMEM((2, 16, D), jnp.float32),
            pltpu.VMEM((2, 16, D), jnp.float32),
            pltpu.VMEM((16, D), jnp.float32),
            pltpu.SemaphoreType.DMA((2,)),
            pltpu.SemaphoreType.DMA((2,)),
        ],
        compiler_params=pltpu.CompilerParams(needs_layout_passes=False),
    )(h_s, pos, rw_f)


def _metadata(counts):
    c = counts.reshape(E)
    nblk = (c + BLK - 1) // BLK
    cumblk = jnp.cumsum(nblk)
    cumblk_excl = cumblk - nblk
    total_blk = cumblk[-1]
    row_offs = cumblk_excl * BLK
    s_idx = jnp.arange(NS, dtype=jnp.int32)
    s_eff = jnp.minimum(s_idx, total_blk - 1)
    step_expert = jnp.sum(
        (s_eff[:, None] >= cumblk[None, :]).astype(jnp.int32), axis=1)
    step_expert = step_expert.astype(jnp.int32)
    step_valid = (s_idx < total_blk).astype(jnp.int32)
    row_offs16 = jnp.zeros((16,), jnp.int32).at[:E].set(row_offs.astype(jnp.int32))
    prev = jnp.concatenate([jnp.full((1,), -1, jnp.int32), step_expert[:-1]])
    run_start = (step_expert != prev).astype(jnp.int32)
    run_id = jnp.cumsum(run_start) - 1
    slot_par = (run_id % 2).astype(jnp.int32)
    diff = (step_expert[None, :] != step_expert[:, None]) & (
        s_idx[None, :] > s_idx[:, None])
    nxt_s = jnp.min(jnp.where(diff, s_idx[None, :], NS), axis=1)
    has_next = (nxt_s < NS).astype(jnp.int32)
    next_expert = step_expert[jnp.minimum(nxt_s, NS - 1)]
    return (step_expert, step_valid, run_start, next_expert, slot_par,
            has_next, row_offs16)


def kernel(hidden_states, gate_w, w_lin, w_v, w_1):
    b, s, d = hidden_states.shape
    x = hidden_states.reshape(T, D)
    logits, rw, sel, rank, counts = _router(x, gate_w)
    (step_expert, step_valid, run_start, next_expert, slot_par, has_next,
     row_offs16) = _metadata(counts)

    sel_f = sel.reshape(-1)
    rank_f = rank.reshape(-1)
    x_s, pos = _dispatch(x, sel_f, rank_f, row_offs16)

    h_s = _grouped(step_expert, step_valid, run_start, next_expert, slot_par,
                   has_next, x_s, w_lin, w_v, w_1)

    out = _combine(h_s, pos, rw.reshape(-1))
    return out.reshape(b, s, d), logits


# speedup vs baseline: 1.4479x; 1.4479x over previous
import functools

import jax
import jax.numpy as jnp
from jax import lax
from jax.experimental import pallas as pl
from jax.experimental.pallas import tpu as pltpu
from jax.experimental.pallas import tpu_sc as plsc

T = 2048
D = 1024
F = 2048
E = 8
K = 2
RB = 256
NRB = T // RB
BLK = 256
FB = 512
NF = F // FB
NS = T * K // BLK + E
P_PAD = NS * BLK


def _router_body(x_ref, gw_ref, logits_ref, rw_ref, sel_ref, rank_ref,
                 counts_ref, carry):
    i = pl.program_id(0)

    @pl.when(i == 0)
    def _():
        carry[...] = jnp.zeros_like(carry)

    x = x_ref[...]
    gw = gw_ref[...]
    logits = lax.dot_general(x, gw, (((1,), (1,)), ((), ())))
    logits_ref[...] = logits
    m = jnp.max(logits, axis=1, keepdims=True)
    ex = jnp.exp(logits - m)
    sm = ex / jnp.sum(ex, axis=1, keepdims=True)
    eio = lax.broadcasted_iota(jnp.int32, (RB, E), 1)
    m0 = jnp.max(sm, axis=1, keepdims=True)
    a0 = jnp.min(jnp.where(sm == m0, eio, E), axis=1, keepdims=True)
    sm1 = jnp.where(eio == a0, -jnp.inf, sm)
    m1 = jnp.max(sm1, axis=1, keepdims=True)
    a1 = jnp.min(jnp.where(sm1 == m1, eio, E), axis=1, keepdims=True)
    oh = (eio == a0).astype(jnp.float32) + (eio == a1).astype(jnp.float32)
    r0 = lax.broadcasted_iota(jnp.int32, (RB, RB), 0)
    r1 = lax.broadcasted_iota(jnp.int32, (RB, RB), 1)
    ltri = (r1 < r0).astype(jnp.float32)
    rank_f = lax.dot_general(ltri, oh, (((1,), (0,)), ((), ()))) + carry[...]
    rank0 = jnp.sum(jnp.where(eio == a0, rank_f, 0.0), axis=1, keepdims=True)
    rank1 = jnp.sum(jnp.where(eio == a1, rank_f, 0.0), axis=1, keepdims=True)
    rw_ref[...] = jnp.concatenate([m0, m1], axis=1)
    sel_ref[...] = jnp.concatenate([a0, a1], axis=1)
    rank_ref[...] = jnp.concatenate([rank0, rank1], axis=1).astype(jnp.int32)
    new_carry = carry[...] + jnp.sum(oh, axis=0, keepdims=True)
    carry[...] = new_carry

    @pl.when(i == NRB - 1)
    def _():
        counts_ref[...] = new_carry.astype(jnp.int32)


@jax.jit
def _router(x, gate_w):
    return pl.pallas_call(
        _router_body,
        grid=(NRB,),
        in_specs=[
            pl.BlockSpec((RB, D), lambda i: (i, 0)),
            pl.BlockSpec((E, D), lambda i: (0, 0)),
        ],
        out_specs=[
            pl.BlockSpec((RB, E), lambda i: (i, 0)),
            pl.BlockSpec((RB, K), lambda i: (i, 0)),
            pl.BlockSpec((RB, K), lambda i: (i, 0)),
            pl.BlockSpec((RB, K), lambda i: (i, 0)),
            pl.BlockSpec((1, E), lambda i: (0, 0)),
        ],
        out_shape=[
            jax.ShapeDtypeStruct((T, E), jnp.float32),
            jax.ShapeDtypeStruct((T, K), jnp.float32),
            jax.ShapeDtypeStruct((T, K), jnp.int32),
            jax.ShapeDtypeStruct((T, K), jnp.int32),
            jax.ShapeDtypeStruct((1, E), jnp.int32),
        ],
        scratch_shapes=[pltpu.VMEM((1, E), jnp.float32)],
    )(x, gate_w)


def _grouped_body(se_ref, valid_ref, rs_ref, ne_ref, sl_ref, hn_ref,
                  x_ref, wl_hbm, wv_hbm, w1_hbm, out_ref,
                  wlb, wvb, w1b, sem_l, sem_v, sem_1):
    s = pl.program_id(0)
    slot = sl_ref[s]

    def _issue(e_idx, slot_idx):
        pltpu.make_async_copy(wl_hbm.at[e_idx], wlb.at[slot_idx],
                              sem_l.at[slot_idx]).start()
        pltpu.make_async_copy(wv_hbm.at[e_idx], wvb.at[slot_idx],
                              sem_v.at[slot_idx]).start()
        pltpu.make_async_copy(w1_hbm.at[e_idx], w1b.at[slot_idx],
                              sem_1.at[slot_idx]).start()

    @pl.when(s == 0)
    def _():
        _issue(se_ref[0], slot)

    @pl.when(rs_ref[s] != 0)
    def _():
        @pl.when(hn_ref[s] != 0)
        def _():
            _issue(ne_ref[s], 1 - slot)

        e = se_ref[s]
        pltpu.make_async_copy(wl_hbm.at[e], wlb.at[slot],
                              sem_l.at[slot]).wait()
        pltpu.make_async_copy(wv_hbm.at[e], wvb.at[slot],
                              sem_v.at[slot]).wait()
        pltpu.make_async_copy(w1_hbm.at[e], w1b.at[slot],
                              sem_1.at[slot]).wait()

    @pl.when(valid_ref[s] != 0)
    def _():
        x = x_ref[...]
        wl = wlb[slot]
        wv = wvb[slot]
        w1 = w1b[slot]
        hl = lax.dot_general(x, wl, (((1,), (1,)), ((), ())))
        hv = lax.dot_general(x, wv, (((1,), (1,)), ((), ())))
        g = (0.5 * hl * (1.0 + lax.erf(hl * 0.7071067811865476))) * hv
        out_ref[...] = lax.dot_general(g, w1, (((1,), (1,)), ((), ())))


@jax.jit
def _grouped(step_expert, step_valid, run_start, next_expert, slot_par,
             has_next, x_s, w_lin, w_v, w_1):
    grid_spec = pltpu.PrefetchScalarGridSpec(
        num_scalar_prefetch=6,
        grid=(NS,),
        in_specs=[
            pl.BlockSpec((BLK, D), lambda s, *_: (s, 0)),
            pl.BlockSpec(memory_space=pl.ANY),
            pl.BlockSpec(memory_space=pl.ANY),
            pl.BlockSpec(memory_space=pl.ANY),
        ],
        out_specs=pl.BlockSpec((BLK, D), lambda s, *_: (s, 0)),
        scratch_shapes=[
            pltpu.VMEM((2, F, D), jnp.float32),
            pltpu.VMEM((2, F, D), jnp.float32),
            pltpu.VMEM((2, D, F), jnp.float32),
            pltpu.SemaphoreType.DMA((2,)),
            pltpu.SemaphoreType.DMA((2,)),
            pltpu.SemaphoreType.DMA((2,)),
        ],
    )
    return pl.pallas_call(
        _grouped_body,
        grid_spec=grid_spec,
        out_shape=jax.ShapeDtypeStruct((P_PAD, D), jnp.float32),
    )(step_expert, step_valid, run_start, next_expert, slot_par, has_next,
      x_s, w_lin, w_v, w_1)


NC = 2
NSUB = 16
NW = NC * NSUB
APW = T * K // NW
CH = 32
NCH = APW // CH
TPW = T // NW

@functools.cache
def _sc_mesh():
    return plsc.VectorSubcoreMesh(core_axis_name="c", subcore_axis_name="s")


def _dispatch_body(x_hbm, sel_hbm, rank_hbm, offs_hbm, xs_hbm, pos_hbm,
                   tok_scr, pos_scr, selv, rankv, offs_v, bufs, semg, sems):
    wid = lax.axis_index("s") * NC + lax.axis_index("c")
    base = wid * APW
    pltpu.sync_copy(offs_hbm, offs_v)
    pltpu.sync_copy(sel_hbm.at[pl.ds(base, APW)], selv)
    pltpu.sync_copy(rank_hbm.at[pl.ds(base, APW)], rankv)
    for j in range(NCH):
        for q in range(CH // 16):
            o = j * CH + q * 16
            sl = selv[pl.ds(o, 16)]
            rk = rankv[pl.ds(o, 16)]
            off = plsc.load_gather(offs_v, [sl])
            pos_scr[j, pl.ds(q * 16, 16)] = off + rk
            tvec = (base + o + lax.iota(jnp.int32, 16)) // K
            tok_scr[j, pl.ds(q * 16, 16)] = tvec
        pltpu.sync_copy(pos_scr.at[j], pos_hbm.at[pl.ds(base + j * CH, CH)])
    gh = [None] * NCH
    sh = [None] * NCH
    gh[0] = pltpu.async_copy(x_hbm.at[tok_scr.at[0]], bufs.at[0], semg.at[0])
    for j in range(NCH):
        b = j % 2
        gh[j].wait()
        sh[j] = pltpu.async_copy(bufs.at[b], xs_hbm.at[pos_scr.at[j]],
                                 sems.at[b])
        if j + 1 < NCH:
            if j >= 1:
                sh[j - 1].wait()
            gh[j + 1] = pltpu.async_copy(x_hbm.at[tok_scr.at[j + 1]],
                                         bufs.at[1 - b], semg.at[1 - b])
    sh[NCH - 2].wait()
    sh[NCH - 1].wait()


@jax.jit
def _dispatch(x, sel_f, rank_f, row_offs16):
    return pl.kernel(
        _dispatch_body,
        mesh=_sc_mesh(),
        out_type=[
            jax.ShapeDtypeStruct((P_PAD, D), jnp.float32),
            jax.ShapeDtypeStruct((T * K,), jnp.int32),
        ],
        scratch_types=[
            pltpu.VMEM((NCH, CH), jnp.int32),
            pltpu.VMEM((NCH, CH), jnp.int32),
            pltpu.VMEM((APW,), jnp.int32),
            pltpu.VMEM((APW,), jnp.int32),
            pltpu.VMEM((16,), jnp.int32),
            pltpu.VMEM((2, CH, D), jnp.float32),
            pltpu.SemaphoreType.DMA((2,)),
            pltpu.SemaphoreType.DMA((2,)),
        ],
        compiler_params=pltpu.CompilerParams(needs_layout_passes=False),
    )(x, sel_f, rank_f, row_offs16)


NCT = TPW // 16


def _combine_body(h_hbm, pos_hbm, rw_hbm, out_hbm,
                  posv, rwv, p0s, p1s, w0s, w1s, w0c, w1c,
                  bufa, bufb, bufo, sema, semb):
    wid = lax.axis_index("s") * NC + lax.axis_index("c")
    tbase = wid * TPW
    ab = tbase * K
    pltpu.sync_copy(pos_hbm.at[pl.ds(ab, TPW * K)], posv)
    pltpu.sync_copy(rw_hbm.at[pl.ds(ab, TPW * K)], rwv)
    ev = lax.iota(jnp.int32, 16) * 2
    od = ev + 1
    for j in range(NCT):
        o = j * 2 * 16
        p0s[j, :] = plsc.load_gather(posv, [o + ev])
        p1s[j, :] = plsc.load_gather(posv, [o + od])
        w0s[j, :] = plsc.load_gather(rwv, [o + ev])
        w1s[j, :] = plsc.load_gather(rwv, [o + od])
    ah = [None] * NCT
    bh = [None] * NCT
    ah[0] = pltpu.async_copy(h_hbm.at[p0s[0, :]], bufa.at[0], sema.at[0])
    bh[0] = pltpu.async_copy(h_hbm.at[p1s[0, :]], bufb.at[0], semb.at[0])
    for j in range(NCT):
        b = j % 2
        ah[j].wait()
        bh[j].wait()
        if j + 1 < NCT:
            ah[j + 1] = pltpu.async_copy(h_hbm.at[p0s[j + 1, :]],
                                         bufa.at[1 - b], sema.at[1 - b])
            bh[j + 1] = pltpu.async_copy(h_hbm.at[p1s[j + 1, :]],
                                         bufb.at[1 - b], semb.at[1 - b])
        w0c[...] = w0s[j, :]
        w1c[...] = w1s[j, :]

        def body(i, carry):
            iv = jnp.broadcast_to(i, (16,)).astype(jnp.int32)
            ws0 = plsc.load_gather(w0c, [iv])
            ws1 = plsc.load_gather(w1c, [iv])
            for q in range(D // 16):
                va = bufa[b, i, pl.ds(q * 16, 16)]
                vb = bufb[b, i, pl.ds(q * 16, 16)]
                bufo[i, pl.ds(q * 16, 16)] = va * ws0 + vb * ws1
            return carry

        lax.fori_loop(0, 16, body, 0)
        pltpu.sync_copy(bufo, out_hbm.at[pl.ds(tbase + j * 16, 16)])


@jax.jit
def _combine(h_s, pos, rw_f):
    return pl.kernel(
        _combine_body,
        mesh=_sc_mesh(),
        out_type=jax.ShapeDtypeStruct((T, D), jnp.float32),
        scratch_types=[
            pltpu.VMEM((TPW * K,), jnp.int32),
            pltpu.VMEM((TPW * K,), jnp.float32),
            pltpu.VMEM((NCT, 16), jnp.int32),
            pltpu.VMEM((NCT, 16), jnp.int32),
            pltpu.VMEM((NCT, 16), jnp.float32),
            pltpu.VMEM((NCT, 16), jnp.float32),
            pltpu.VMEM((16,), jnp.float32),
            pltpu.VMEM((16,), jnp.float32),
            pltpu.VMEM((2, 16, D), jnp.float32),
            pltpu.VMEM((2, 16, D), jnp.float32),
            pltpu.VMEM((16, D), jnp.float32),
            pltpu.SemaphoreType.DMA((2,)),
            pltpu.SemaphoreType.DMA((2,)),
        ],
        compiler_params=pltpu.CompilerParams(needs_layout_passes=False),
    )(h_s, pos, rw_f)


def _metadata(counts):
    c = counts.reshape(E)
    nblk = (c + BLK - 1) // BLK
    cumblk = jnp.cumsum(nblk)
    cumblk_excl = cumblk - nblk
    total_blk = cumblk[-1]
    row_offs = cumblk_excl * BLK
    s_idx = jnp.arange(NS, dtype=jnp.int32)
    s_eff = jnp.minimum(s_idx, total_blk - 1)
    step_expert = jnp.sum(
        (s_eff[:, None] >= cumblk[None, :]).astype(jnp.int32), axis=1)
    step_expert = step_expert.astype(jnp.int32)
    step_valid = (s_idx < total_blk).astype(jnp.int32)
    row_offs16 = jnp.zeros((16,), jnp.int32).at[:E].set(row_offs.astype(jnp.int32))
    prev = jnp.concatenate([jnp.full((1,), -1, jnp.int32), step_expert[:-1]])
    run_start = (step_expert != prev).astype(jnp.int32)
    run_id = jnp.cumsum(run_start) - 1
    slot_par = (run_id % 2).astype(jnp.int32)
    diff = (step_expert[None, :] != step_expert[:, None]) & (
        s_idx[None, :] > s_idx[:, None])
    nxt_s = jnp.min(jnp.where(diff, s_idx[None, :], NS), axis=1)
    has_next = (nxt_s < NS).astype(jnp.int32)
    next_expert = step_expert[jnp.minimum(nxt_s, NS - 1)]
    return (step_expert, step_valid, run_start, next_expert, slot_par,
            has_next, row_offs16)


def kernel(hidden_states, gate_w, w_lin, w_v, w_1):
    b, s, d = hidden_states.shape
    x = hidden_states.reshape(T, D)
    logits, rw, sel, rank, counts = _router(x, gate_w)
    (step_expert, step_valid, run_start, next_expert, slot_par, has_next,
     row_offs16) = _metadata(counts)

    sel_f = sel.reshape(-1)
    rank_f = rank.reshape(-1)
    x_s, pos = _dispatch(x, sel_f, rank_f, row_offs16)

    h_s = _grouped(step_expert, step_valid, run_start, next_expert, slot_par,
                   has_next, x_s, w_lin, w_v, w_1)

    out = _combine(h_s, pos, rw.reshape(-1))
    return out.reshape(b, s, d), logits
